# Initial kernel scaffold; baseline (speedup 1.0000x reference)
import jax
import jax.numpy as jnp
from jax import lax
from jax.experimental import pallas as pl
from jax.experimental.pallas import tpu as pltpu
from jax.experimental.pallas import tpu_sc as plsc

N, E, D, G, H_FC = 10000, 320000, 128, 64, 64
NC, NS = 2, 16
NW = NC * NS
EP = E // NW
CH = 80
NCH = EP // CH
SZ = N // NS
BN = 1000
GRID = N // BN


def _make_seg_kernel(with_counts):
    outs = [jax.ShapeDtypeStruct((NC, N, D), jnp.float32)]
    scratch = [
        pltpu.VMEM((NCH, CH), jnp.int32),
        pltpu.VMEM((NCH, CH), jnp.int32),
        pltpu.VMEM((CH, D), jnp.float32),
        pltpu.VMEM_SHARED((N, D), jnp.float32),
        pltpu.SemaphoreType.DMA,
    ]
    if with_counts:
        outs.append(jax.ShapeDtypeStruct((NC, N, 16), jnp.float32))
        scratch += [
            pltpu.VMEM((CH, 16), jnp.float32),
            pltpu.VMEM_SHARED((N, 16), jnp.float32),
        ]
    mesh = plsc.VectorSubcoreMesh(core_axis_name="c", subcore_axis_name="s")

    def body(table_hbm, e4_hbm, *rest):
        if with_counts:
            (s_out, cnt_out, src_i, dst_i, rows, acc, sem, ones16, cnt_sh) = rest
        else:
            (s_out, src_i, dst_i, rows, acc, sem) = rest
        c = lax.axis_index("c")
        s = lax.axis_index("s")
        wid = s * NC + c
        base = s * SZ

        pltpu.sync_copy(e4_hbm.at[wid], src_i)
        pltpu.sync_copy(e4_hbm.at[NW + wid], dst_i)

        def zr(i, _):
            rows[i // 8, pl.ds((i % 8) * 16, 16)] = jnp.zeros((16,), jnp.float32)
            return 0
        lax.fori_loop(0, CH * (D // 16), zr, 0)
        nfull = SZ // CH
        rem = SZ - nfull * CH
        for j in range(nfull):
            pltpu.sync_copy(rows, acc.at[pl.ds(base + j * CH, CH)])
        if rem:
            pltpu.sync_copy(rows.at[pl.ds(0, rem)],
                            acc.at[pl.ds(base + nfull * CH, rem)])
        if with_counts:
            def zo(i, _):
                ones16[i, pl.ds(0, 16)] = jnp.zeros((16,), jnp.float32)
                return 0
            lax.fori_loop(0, CH, zo, 0)
            for j in range(nfull):
                pltpu.sync_copy(ones16, cnt_sh.at[pl.ds(base + j * CH, CH)])
            if rem:
                pltpu.sync_copy(ones16.at[pl.ds(0, rem)],
                                cnt_sh.at[pl.ds(base + nfull * CH, rem)])
            def fo(i, _):
                ones16[i, pl.ds(0, 16)] = jnp.ones((16,), jnp.float32)
                return 0
            lax.fori_loop(0, CH, fo, 0)
        plsc.subcore_barrier()

        def step(j, _):
            pltpu.async_copy(table_hbm.at[src_i.at[j]], rows, sem).wait()
            pltpu.sync_copy(rows, acc.at[dst_i.at[j]], add=True)
            if with_counts:
                pltpu.sync_copy(ones16, cnt_sh.at[dst_i.at[j]], add=True)
            return 0
        lax.fori_loop(0, NCH, step, 0)
        plsc.subcore_barrier()

        pltpu.sync_copy(acc.at[pl.ds(base, SZ)], s_out.at[c, pl.ds(base, SZ)])
        if with_counts:
            pltpu.sync_copy(cnt_sh.at[pl.ds(base, SZ)],
                            cnt_out.at[c, pl.ds(base, SZ)])

    return pl.kernel(body, out_type=outs, mesh=mesh, scratch_types=scratch)


_seg_with_cnt = _make_seg_kernel(True)
_seg_plain = _make_seg_kernel(False)


def _sage_body(s_ref, c_ref, x_ref, wl_ref, bl_ref, wr_ref, o_ref):
    ssum = s_ref[0] + s_ref[1]
    cnt = c_ref[0, :, 0:1] + c_ref[1, :, 0:1]
    agg = ssum * (1.0 / jnp.maximum(cnt, 1.0))
    h = (lax.dot_general(agg, wl_ref[...], (((1,), (1,)), ((), ())),
                         preferred_element_type=jnp.float32)
         + bl_ref[...]
         + lax.dot_general(x_ref[...], wr_ref[...], (((1,), (1,)), ((), ())),
                           preferred_element_type=jnp.float32))
    o_ref[...] = jnp.maximum(h, 0.0)


_sage_tc = pl.pallas_call(
    _sage_body,
    grid=(GRID,),
    in_specs=[
        pl.BlockSpec((NC, BN, D), lambda i: (0, i, 0)),
        pl.BlockSpec((NC, BN, 16), lambda i: (0, i, 0)),
        pl.BlockSpec((BN, D), lambda i: (i, 0)),
        pl.BlockSpec((D, D), lambda i: (0, 0)),
        pl.BlockSpec((1, D), lambda i: (0, 0)),
        pl.BlockSpec((D, D), lambda i: (0, 0)),
    ],
    out_specs=pl.BlockSpec((BN, D), lambda i: (i, 0)),
    out_shape=jax.ShapeDtypeStruct((N, D), jnp.float32),
)


def _head_body(s_ref, c_ref, h1_ref, b_ref, wl_ref, bl_ref, wr_ref,
               wf1_ref, bf1_ref, wf2_ref, bf2_ref, o_ref,
               pooled_s, gcnt_s):
    i = pl.program_id(0)

    @pl.when(i == 0)
    def _():
        pooled_s[...] = jnp.zeros_like(pooled_s)
        gcnt_s[...] = jnp.zeros_like(gcnt_s)

    ssum = s_ref[0] + s_ref[1]
    cnt = c_ref[0, :, 0:1] + c_ref[1, :, 0:1]
    agg = ssum * (1.0 / jnp.maximum(cnt, 1.0))
    h2 = (lax.dot_general(agg, wl_ref[...], (((1,), (1,)), ((), ())),
                          preferred_element_type=jnp.float32)
          + bl_ref[...]
          + lax.dot_general(h1_ref[...], wr_ref[...], (((1,), (1,)), ((), ())),
                            preferred_element_type=jnp.float32))
    h2 = jnp.maximum(h2, 0.0)

    b = b_ref[0, 0]
    gids = lax.broadcasted_iota(jnp.int32, (BN, G), 1)
    onehot = (b[:, None] == gids).astype(jnp.float32)
    pooled_s[...] += lax.dot_general(onehot, h2, (((0,), (0,)), ((), ())),
                                     preferred_element_type=jnp.float32)
    gcnt_s[...] += lax.dot_general(onehot, jnp.ones((BN, 8), jnp.float32),
                                   (((0,), (0,)), ((), ())),
                                   preferred_element_type=jnp.float32)

    @pl.when(i == GRID - 1)
    def _():
        pooled = pooled_s[...] * (1.0 / jnp.maximum(gcnt_s[:, 0:1], 1.0))
        t = jnp.maximum(
            lax.dot_general(pooled, wf1_ref[...], (((1,), (1,)), ((), ())),
                            preferred_element_type=jnp.float32)
            + bf1_ref[...], 0.0)
        o_ref[...] = (lax.dot_general(t, wf2_ref[...], (((1,), (1,)), ((), ())),
                                      preferred_element_type=jnp.float32)
                      + bf2_ref[...])


_head_tc = pl.pallas_call(
    _head_body,
    grid=(GRID,),
    in_specs=[
        pl.BlockSpec((NC, BN, D), lambda i: (0, i, 0)),
        pl.BlockSpec((NC, BN, 16), lambda i: (0, i, 0)),
        pl.BlockSpec((BN, D), lambda i: (i, 0)),
        pl.BlockSpec((1, 1, BN), lambda i: (i, 0, 0)),
        pl.BlockSpec((D, D), lambda i: (0, 0)),
        pl.BlockSpec((1, D), lambda i: (0, 0)),
        pl.BlockSpec((D, D), lambda i: (0, 0)),
        pl.BlockSpec((H_FC, D), lambda i: (0, 0)),
        pl.BlockSpec((1, H_FC), lambda i: (0, 0)),
        pl.BlockSpec((1, H_FC), lambda i: (0, 0)),
        pl.BlockSpec((1, 1), lambda i: (0, 0)),
    ],
    out_specs=pl.BlockSpec((G, 1), lambda i: (0, 0)),
    out_shape=jax.ShapeDtypeStruct((G, 1), jnp.float32),
    scratch_shapes=[
        pltpu.VMEM((G, D), jnp.float32),
        pltpu.VMEM((G, 8), jnp.float32),
    ],
)


def kernel(x, edge_index, batch, W1l, b1l, W1r, W2l, b2l, W2r,
           Wf1, bf1, Wf2, bf2):
    e4 = edge_index.reshape(2 * NW, NCH, CH)
    s1, cnt16 = _seg_with_cnt(x, e4)
    h1 = _sage_tc(s1, cnt16, x, W1l, b1l.reshape(1, D), W1r)
    s2 = _seg_plain(h1, e4)
    out = _head_tc(s2, cnt16, h1, batch.reshape(GRID, 1, BN),
                   W2l, b2l.reshape(1, D), W2r,
                   Wf1, bf1.reshape(1, H_FC), Wf2, bf2.reshape(1, 1))
    return out.reshape(-1)


# speedup vs baseline: 3.5276x; 3.5276x over previous
import functools

import jax
import jax.numpy as jnp
from jax import lax
from jax.experimental import pallas as pl
from jax.experimental.pallas import tpu as pltpu
from jax.experimental.pallas import tpu_sc as plsc

N, E, D, G, H_FC = 10000, 320000, 128, 64, 64
NC, NS = 2, 16
NW = NC * NS
EP = E // NW
CH = 80
NCH = EP // CH
NBUF = 25
NG = NCH // NBUF
NP = 10240
SZ = NP // NS
RP = NP // NW
BN = 1000
GRID = N // BN


def _make_seg_kernel():
    outs = [jax.ShapeDtypeStruct((NC, NP, D), jnp.float32)]
    scratch = [
        pltpu.VMEM((NBUF, CH), jnp.int32),
        pltpu.VMEM((NBUF, CH), jnp.int32),
        pltpu.VMEM((CH, D), jnp.float32),
        pltpu.VMEM_SHARED((NP, D), jnp.float32),
        pltpu.SemaphoreType.DMA,
    ]
    mesh = plsc.VectorSubcoreMesh(core_axis_name="c", subcore_axis_name="s",
                                  num_cores=NC, num_subcores=NS)

    def body(table_hbm, e4_hbm, *rest):
        (s_out, src_i, dst_i, rows, acc, sem) = rest
        c = lax.axis_index("c")
        s = lax.axis_index("s")
        wid = s * NC + c
        base = s * SZ

        def zr(i, _):
            rows[i // 8, pl.ds((i % 8) * 16, 16)] = jnp.zeros((16,), jnp.float32)
            return 0
        lax.fori_loop(0, CH * (D // 16), zr, 0)
        nfull = SZ // CH
        rem = SZ - nfull * CH
        for j in range(nfull):
            pltpu.sync_copy(rows, acc.at[pl.ds(base + j * CH, CH)])
        if rem:
            pltpu.sync_copy(rows.at[pl.ds(0, rem)],
                            acc.at[pl.ds(base + nfull * CH, rem)])
        plsc.subcore_barrier()

        def group(g, _):
            pltpu.sync_copy(e4_hbm.at[wid * NG + g], src_i)
            pltpu.sync_copy(e4_hbm.at[(NW + wid) * NG + g], dst_i)

            def step(j, _):
                pltpu.async_copy(table_hbm.at[src_i.at[j]], rows, sem).wait()
                pltpu.sync_copy(rows, acc.at[dst_i.at[j]], add=True)
                return 0
            lax.fori_loop(0, NBUF, step, 0)
            return 0
        lax.fori_loop(0, NG, group, 0)

        plsc.subcore_barrier()

        pltpu.sync_copy(acc.at[pl.ds(base, SZ)], s_out.at[c, pl.ds(base, SZ)])

    return pl.kernel(body, out_type=outs[0], mesh=mesh,
                     scratch_types=scratch)


SCAN = 2000


def _make_hist_kernel():
    out = jax.ShapeDtypeStruct((NP * D,), jnp.float32)
    scratch = [
        pltpu.VMEM((SCAN,), jnp.int32),
        pltpu.VMEM((RP + 16,), jnp.float32),
        pltpu.VMEM((16 * D,), jnp.float32),
    ]
    mesh = plsc.VectorSubcoreMesh(core_axis_name="c", subcore_axis_name="s",
                                  num_cores=NC, num_subcores=NS)

    def body(dst_hbm, cnt_out, dbuf, hist, obuf):
        c = lax.axis_index("c")
        s = lax.axis_index("s")
        wid = s * NC + c
        lo = wid * RP
        ones_v = jnp.ones((16,), jnp.float32)

        def zh(i, _):
            hist[pl.ds(i * 16, 16)] = jnp.zeros((16,), jnp.float32)
            return 0
        lax.fori_loop(0, (RP + 16) // 16, zh, 0)

        def hgroup(g, _):
            pltpu.sync_copy(dst_hbm.at[pl.ds(g * SCAN, SCAN)], dbuf)

            def hstep(j, _):
                v = dbuf[pl.ds(j * 16, 16)]
                rel = v - lo
                oob = (rel < 0) | (rel >= RP)
                rel = jnp.where(oob, RP, rel)
                plsc.addupdate_scatter(hist, [rel], ones_v)
                return 0
            lax.fori_loop(0, SCAN // 16, hstep, 0)
            return 0
        lax.fori_loop(0, E // SCAN, hgroup, 0)

        def ochunk(t, _):
            for rr in range(16):
                row = plsc.load_gather(
                    hist, [jnp.full((16,), t * 16 + rr, jnp.int32)])
                for q in range(D // 16):
                    obuf[pl.ds(rr * D + q * 16, 16)] = row
            pltpu.sync_copy(obuf, cnt_out.at[pl.ds((lo + t * 16) * D, 16 * D)])
            return 0
        lax.fori_loop(0, RP // 16, ochunk, 0)

    return pl.kernel(
        body, out_type=out, mesh=mesh, scratch_types=scratch,
        compiler_params=pltpu.CompilerParams(needs_layout_passes=False))


@functools.lru_cache(maxsize=None)
def _seg_kernel():
    return _make_seg_kernel()


@functools.lru_cache(maxsize=None)
def _hist_kernel():
    return _make_hist_kernel()


def _sage_body(s_ref, c_ref, x_ref, wl_ref, bl_ref, wr_ref, o_ref):
    ssum = s_ref[0] + s_ref[1]
    agg = ssum * (1.0 / jnp.maximum(c_ref[...], 1.0))
    h = (lax.dot_general(agg, wl_ref[...], (((1,), (1,)), ((), ())),
                         preferred_element_type=jnp.float32)
         + bl_ref[...]
         + lax.dot_general(x_ref[...], wr_ref[...], (((1,), (1,)), ((), ())),
                           preferred_element_type=jnp.float32))
    o_ref[...] = jnp.maximum(h, 0.0)


_sage_tc = pl.pallas_call(
    _sage_body,
    grid=(GRID,),
    in_specs=[
        pl.BlockSpec((NC, BN, D), lambda i: (0, i, 0)),
        pl.BlockSpec((BN, D), lambda i: (i, 0)),
        pl.BlockSpec((BN, D), lambda i: (i, 0)),
        pl.BlockSpec((D, D), lambda i: (0, 0)),
        pl.BlockSpec((1, D), lambda i: (0, 0)),
        pl.BlockSpec((D, D), lambda i: (0, 0)),
    ],
    out_specs=pl.BlockSpec((BN, D), lambda i: (i, 0)),
    out_shape=jax.ShapeDtypeStruct((N, D), jnp.float32),
)


def _head_body(s_ref, c_ref, h1_ref, b_ref, wl_ref, bl_ref, wr_ref,
               wf1_ref, bf1_ref, wf2_ref, bf2_ref, o_ref,
               pooled_s, gcnt_s):
    i = pl.program_id(0)

    @pl.when(i == 0)
    def _():
        pooled_s[...] = jnp.zeros_like(pooled_s)
        gcnt_s[...] = jnp.zeros_like(gcnt_s)

    ssum = s_ref[0] + s_ref[1]
    agg = ssum * (1.0 / jnp.maximum(c_ref[...], 1.0))
    h2 = (lax.dot_general(agg, wl_ref[...], (((1,), (1,)), ((), ())),
                          preferred_element_type=jnp.float32)
          + bl_ref[...]
          + lax.dot_general(h1_ref[...], wr_ref[...], (((1,), (1,)), ((), ())),
                            preferred_element_type=jnp.float32))
    h2 = jnp.maximum(h2, 0.0)

    b = b_ref[0, 0]
    gids = lax.broadcasted_iota(jnp.int32, (BN, G), 1)
    onehot = (b[:, None] == gids).astype(jnp.float32)
    pooled_s[...] += lax.dot_general(onehot, h2, (((0,), (0,)), ((), ())),
                                     preferred_element_type=jnp.float32)
    gcnt_s[...] += lax.dot_general(onehot, jnp.ones((BN, D), jnp.float32),
                                   (((0,), (0,)), ((), ())),
                                   preferred_element_type=jnp.float32)

    @pl.when(i == GRID - 1)
    def _():
        pooled = pooled_s[...] * (1.0 / jnp.maximum(gcnt_s[...], 1.0))
        t = jnp.maximum(
            lax.dot_general(pooled, wf1_ref[...], (((1,), (1,)), ((), ())),
                            preferred_element_type=jnp.float32)
            + bf1_ref[...], 0.0)
        o_ref[...] = (lax.dot_general(wf2_ref[...], t, (((1,), (1,)), ((), ())),
                                      preferred_element_type=jnp.float32)
                      + bf2_ref[0, 0])


_head_tc = pl.pallas_call(
    _head_body,
    grid=(GRID,),
    in_specs=[
        pl.BlockSpec((NC, BN, D), lambda i: (0, i, 0)),
        pl.BlockSpec((BN, D), lambda i: (i, 0)),
        pl.BlockSpec((BN, D), lambda i: (i, 0)),
        pl.BlockSpec((1, 1, BN), lambda i: (i, 0, 0)),
        pl.BlockSpec((D, D), lambda i: (0, 0)),
        pl.BlockSpec((1, D), lambda i: (0, 0)),
        pl.BlockSpec((D, D), lambda i: (0, 0)),
        pl.BlockSpec((H_FC, D), lambda i: (0, 0)),
        pl.BlockSpec((1, H_FC), lambda i: (0, 0)),
        pl.BlockSpec((1, H_FC), lambda i: (0, 0)),
        pl.BlockSpec((1, 1), lambda i: (0, 0)),
    ],
    out_specs=pl.BlockSpec((1, G), lambda i: (0, 0)),
    out_shape=jax.ShapeDtypeStruct((1, G), jnp.float32),
    scratch_shapes=[
        pltpu.VMEM((G, D), jnp.float32),
        pltpu.VMEM((G, D), jnp.float32),
    ],
)


def kernel(x, edge_index, batch, W1l, b1l, W1r, W2l, b2l, W2r,
           Wf1, bf1, Wf2, bf2):
    e4 = edge_index.reshape(2 * NW * NG, NBUF, CH)
    cnt = _hist_kernel()(edge_index[1]).reshape(NP, D)
    s1 = _seg_kernel()(x, e4)
    h1 = _sage_tc(s1, cnt, x, W1l, b1l.reshape(1, D), W1r)
    s2 = _seg_kernel()(h1, e4)
    out = _head_tc(s2, cnt, h1, batch.reshape(GRID, 1, BN),
                   W2l, b2l.reshape(1, D), W2r,
                   Wf1, bf1.reshape(1, H_FC), Wf2, bf2.reshape(1, 1))
    return out.reshape(-1)


# speedup vs baseline: 11.3133x; 3.2071x over previous
import functools

import jax
import jax.numpy as jnp
from jax import lax
from jax.experimental import pallas as pl
from jax.experimental.pallas import tpu as pltpu
from jax.experimental.pallas import tpu_sc as plsc

N, E, D, G, H_FC = 10000, 320000, 128, 64, 64
NC, NS = 2, 16
NW = NC * NS
EP = E // NW
CH = 125
NCH = EP // CH
NBUF = 16
NG = NCH // NBUF
NP = 10240
SZ = NP // NS
RP = NP // NW
BN = 1000
GRID = N // BN


def _make_seg_kernel():
    outs = [jax.ShapeDtypeStruct((NC, NP, D), jnp.float32)]
    scratch = [
        pltpu.VMEM((NBUF, CH), jnp.int32),
        pltpu.VMEM((NBUF, CH), jnp.int32),
        pltpu.VMEM((CH, D), jnp.float32),
        pltpu.VMEM((CH, D), jnp.float32),
        pltpu.VMEM_SHARED((NP, D), jnp.float32),
        pltpu.SemaphoreType.DMA,
        pltpu.SemaphoreType.DMA,
    ]
    mesh = plsc.VectorSubcoreMesh(core_axis_name="c", subcore_axis_name="s",
                                  num_cores=NC, num_subcores=NS)

    def body(table_hbm, e4_hbm, *rest):
        (s_out, src_i, dst_i, rows0, rows1, acc, sem0, sem1) = rest
        c = lax.axis_index("c")
        s = lax.axis_index("s")
        wid = s * NC + c
        base = s * SZ

        def zr(i, _):
            rows0[i // 8, pl.ds((i % 8) * 16, 16)] = jnp.zeros((16,),
                                                               jnp.float32)
            return 0
        lax.fori_loop(0, CH * (D // 16), zr, 0)
        for j in range(SZ // 80):
            pltpu.sync_copy(rows0.at[pl.ds(0, 80)],
                            acc.at[pl.ds(base + j * 80, 80)])
        plsc.subcore_barrier()

        def gather(j, rbuf, sem):
            return pltpu.async_copy(table_hbm.at[src_i.at[j]], rbuf, sem)

        def group(g, _):
            pltpu.sync_copy(e4_hbm.at[wid * NG + g], src_i)
            pltpu.sync_copy(e4_hbm.at[(NW + wid) * NG + g], dst_i)
            gather(0, rows0, sem0)

            def pair(t, _):
                j0 = t * 2
                gather(j0 + 1, rows1, sem1)
                pltpu.make_async_copy(table_hbm.at[src_i.at[j0]],
                                      rows0, sem0).wait()
                pltpu.sync_copy(rows0, acc.at[dst_i.at[j0]], add=True)

                @pl.when(t < NBUF // 2 - 1)
                def _():
                    gather(j0 + 2, rows0, sem0)
                pltpu.make_async_copy(table_hbm.at[src_i.at[j0 + 1]],
                                      rows1, sem1).wait()
                pltpu.sync_copy(rows1, acc.at[dst_i.at[j0 + 1]], add=True)
                return 0
            lax.fori_loop(0, NBUF // 2, pair, 0)
            return 0
        lax.fori_loop(0, NG, group, 0)

        plsc.subcore_barrier()

        pltpu.sync_copy(acc.at[pl.ds(base, SZ)], s_out.at[c, pl.ds(base, SZ)])

    return pl.kernel(body, out_type=outs[0], mesh=mesh,
                     scratch_types=scratch)


SCAN = 2000


def _make_hist_kernel():
    out = jax.ShapeDtypeStruct((NC * NP * D,), jnp.float32)
    scratch = [
        pltpu.VMEM((SCAN,), jnp.int32),
        pltpu.VMEM((NP,), jnp.float32),
        pltpu.VMEM((16 * SZ,), jnp.float32),
        pltpu.VMEM((16 * D,), jnp.float32),
        pltpu.VMEM_SHARED((NS * NP,), jnp.float32),
    ]
    mesh = plsc.VectorSubcoreMesh(core_axis_name="c", subcore_axis_name="s",
                                  num_cores=NC, num_subcores=NS)

    def body(dst_hbm, cnt_out, dbuf, hist, cbuf, obuf, stage):
        c = lax.axis_index("c")
        s = lax.axis_index("s")
        wid = s * NC + c
        ones_v = jnp.ones((16,), jnp.float32)

        def zh(i, _):
            hist[pl.ds(i * 16, 16)] = jnp.zeros((16,), jnp.float32)
            return 0
        lax.fori_loop(0, NP // 16, zh, 0)

        def hgroup(g, _):
            pltpu.sync_copy(dst_hbm.at[pl.ds(wid * EP + g * SCAN, SCAN)],
                            dbuf)

            def hstep(j, _):
                v = dbuf[pl.ds(j * 16, 16)]
                plsc.addupdate_scatter(hist, [v], ones_v)
                return 0
            lax.fori_loop(0, SCAN // 16, hstep, 0)
            return 0
        lax.fori_loop(0, EP // SCAN, hgroup, 0)

        pltpu.sync_copy(hist, stage.at[pl.ds(s * NP, NP)])
        plsc.subcore_barrier()

        lo = s * SZ
        for r in range(NS):
            pltpu.sync_copy(stage.at[pl.ds(r * NP + lo, SZ)],
                            cbuf.at[pl.ds(r * SZ, SZ)])

        def csum(m, _):
            acc16 = cbuf[pl.ds(m * 16, 16)]
            for r in range(1, NS):
                acc16 += cbuf[pl.ds(r * SZ + m * 16, 16)]
            hist[pl.ds(m * 16, 16)] = acc16
            return 0
        lax.fori_loop(0, SZ // 16, csum, 0)

        obase = c * NP * D + lo * D

        def ochunk(t, _):
            for rr in range(16):
                row = plsc.load_gather(
                    hist, [jnp.full((16,), t * 16 + rr, jnp.int32)])
                for q in range(D // 16):
                    obuf[pl.ds(rr * D + q * 16, 16)] = row
            pltpu.sync_copy(obuf, cnt_out.at[pl.ds(obase + t * 16 * D,
                                                   16 * D)])
            return 0
        lax.fori_loop(0, SZ // 16, ochunk, 0)

    return pl.kernel(
        body, out_type=out, mesh=mesh, scratch_types=scratch,
        compiler_params=pltpu.CompilerParams(needs_layout_passes=False))


@functools.lru_cache(maxsize=None)
def _seg_kernel():
    return _make_seg_kernel()


@functools.lru_cache(maxsize=None)
def _hist_kernel():
    return _make_hist_kernel()


def _sage_body(s_ref, c_ref, x_ref, wl_ref, bl_ref, wr_ref, o_ref):
    ssum = s_ref[0] + s_ref[1]
    agg = ssum * (1.0 / jnp.maximum(c_ref[0] + c_ref[1], 1.0))
    h = (lax.dot_general(agg, wl_ref[...], (((1,), (1,)), ((), ())),
                         preferred_element_type=jnp.float32)
         + bl_ref[...]
         + lax.dot_general(x_ref[...], wr_ref[...], (((1,), (1,)), ((), ())),
                           preferred_element_type=jnp.float32))
    o_ref[...] = jnp.maximum(h, 0.0)


_sage_tc = pl.pallas_call(
    _sage_body,
    grid=(GRID,),
    in_specs=[
        pl.BlockSpec((NC, BN, D), lambda i: (0, i, 0)),
        pl.BlockSpec((NC, BN, D), lambda i: (0, i, 0)),
        pl.BlockSpec((BN, D), lambda i: (i, 0)),
        pl.BlockSpec((D, D), lambda i: (0, 0)),
        pl.BlockSpec((1, D), lambda i: (0, 0)),
        pl.BlockSpec((D, D), lambda i: (0, 0)),
    ],
    out_specs=pl.BlockSpec((BN, D), lambda i: (i, 0)),
    out_shape=jax.ShapeDtypeStruct((N, D), jnp.float32),
)


def _head_body(s_ref, c_ref, h1_ref, b_ref, wl_ref, bl_ref, wr_ref,
               wf1_ref, bf1_ref, wf2_ref, bf2_ref, o_ref,
               pooled_s, gcnt_s):
    i = pl.program_id(0)

    @pl.when(i == 0)
    def _():
        pooled_s[...] = jnp.zeros_like(pooled_s)
        gcnt_s[...] = jnp.zeros_like(gcnt_s)

    ssum = s_ref[0] + s_ref[1]
    agg = ssum * (1.0 / jnp.maximum(c_ref[0] + c_ref[1], 1.0))
    h2 = (lax.dot_general(agg, wl_ref[...], (((1,), (1,)), ((), ())),
                          preferred_element_type=jnp.float32)
          + bl_ref[...]
          + lax.dot_general(h1_ref[...], wr_ref[...], (((1,), (1,)), ((), ())),
                            preferred_element_type=jnp.float32))
    h2 = jnp.maximum(h2, 0.0)

    b = b_ref[0, 0]
    gids = lax.broadcasted_iota(jnp.int32, (BN, G), 1)
    onehot = (b[:, None] == gids).astype(jnp.float32)
    pooled_s[...] += lax.dot_general(onehot, h2, (((0,), (0,)), ((), ())),
                                     preferred_element_type=jnp.float32)
    gcnt_s[...] += lax.dot_general(onehot, jnp.ones((BN, D), jnp.float32),
                                   (((0,), (0,)), ((), ())),
                                   preferred_element_type=jnp.float32)

    @pl.when(i == GRID - 1)
    def _():
        pooled = pooled_s[...] * (1.0 / jnp.maximum(gcnt_s[...], 1.0))
        t = jnp.maximum(
            lax.dot_general(pooled, wf1_ref[...], (((1,), (1,)), ((), ())),
                            preferred_element_type=jnp.float32)
            + bf1_ref[...], 0.0)
        o_ref[...] = (lax.dot_general(wf2_ref[...], t, (((1,), (1,)), ((), ())),
                                      preferred_element_type=jnp.float32)
                      + bf2_ref[0, 0])


_head_tc = pl.pallas_call(
    _head_body,
    grid=(GRID,),
    in_specs=[
        pl.BlockSpec((NC, BN, D), lambda i: (0, i, 0)),
        pl.BlockSpec((NC, BN, D), lambda i: (0, i, 0)),
        pl.BlockSpec((BN, D), lambda i: (i, 0)),
        pl.BlockSpec((1, 1, BN), lambda i: (i, 0, 0)),
        pl.BlockSpec((D, D), lambda i: (0, 0)),
        pl.BlockSpec((1, D), lambda i: (0, 0)),
        pl.BlockSpec((D, D), lambda i: (0, 0)),
        pl.BlockSpec((H_FC, D), lambda i: (0, 0)),
        pl.BlockSpec((1, H_FC), lambda i: (0, 0)),
        pl.BlockSpec((1, H_FC), lambda i: (0, 0)),
        pl.BlockSpec((1, 1), lambda i: (0, 0)),
    ],
    out_specs=pl.BlockSpec((1, G), lambda i: (0, 0)),
    out_shape=jax.ShapeDtypeStruct((1, G), jnp.float32),
    scratch_shapes=[
        pltpu.VMEM((G, D), jnp.float32),
        pltpu.VMEM((G, D), jnp.float32),
    ],
)


def kernel(x, edge_index, batch, W1l, b1l, W1r, W2l, b2l, W2r,
           Wf1, bf1, Wf2, bf2):
    e4 = edge_index.reshape(2 * NW * NG, NBUF, CH)
    cnt = _hist_kernel()(edge_index[1]).reshape(NC, NP, D)
    s1 = _seg_kernel()(x, e4)
    h1 = _sage_tc(s1, cnt, x, W1l, b1l.reshape(1, D), W1r)
    s2 = _seg_kernel()(h1, e4)
    out = _head_tc(s2, cnt, h1, batch.reshape(GRID, 1, BN),
                   W2l, b2l.reshape(1, D), W2r,
                   Wf1, bf1.reshape(1, H_FC), Wf2, bf2.reshape(1, 1))
    return out.reshape(-1)


# speedup vs baseline: 11.3142x; 1.0001x over previous
import functools

import jax
import jax.numpy as jnp
from jax import lax
from jax.experimental import pallas as pl
from jax.experimental.pallas import tpu as pltpu
from jax.experimental.pallas import tpu_sc as plsc

N, E, D, G, H_FC = 10000, 320000, 128, 64, 64
NC, NS = 2, 16
NW = NC * NS
EP = E // NW
CH = 125
NCH = EP // CH
NBUF = 16
NG = NCH // NBUF
NP = 10240
SZ = NP // NS
RP = NP // NW
BN = 1000
GRID = N // BN


def _make_seg_kernel():
    outs = [jax.ShapeDtypeStruct((NC, NP, D), jnp.float32)]
    scratch = [
        pltpu.VMEM((NBUF, CH), jnp.int32),
        pltpu.VMEM((NBUF, CH), jnp.int32),
        pltpu.VMEM((CH, D), jnp.float32),
        pltpu.VMEM((CH, D), jnp.float32),
        pltpu.VMEM_SHARED((NP, D), jnp.float32),
        pltpu.SemaphoreType.DMA,
        pltpu.SemaphoreType.DMA,
    ]
    mesh = plsc.VectorSubcoreMesh(core_axis_name="c", subcore_axis_name="s",
                                  num_cores=NC, num_subcores=NS)

    def body(table_hbm, e4_hbm, *rest):
        (s_out, src_i, dst_i, rows0, rows1, acc, sem0, sem1) = rest
        c = lax.axis_index("c")
        s = lax.axis_index("s")
        wid = s * NC + c
        base = s * SZ

        def zr(i, _):
            rows0[i // 8, pl.ds((i % 8) * 16, 16)] = jnp.zeros((16,),
                                                               jnp.float32)
            return 0
        lax.fori_loop(0, CH * (D // 16), zr, 0)
        for j in range(SZ // 80):
            pltpu.sync_copy(rows0.at[pl.ds(0, 80)],
                            acc.at[pl.ds(base + j * 80, 80)])
        plsc.subcore_barrier()

        def gather(j, rbuf, sem):
            return pltpu.async_copy(table_hbm.at[src_i.at[j]], rbuf, sem)

        def group(g, _):
            pltpu.sync_copy(e4_hbm.at[wid * NG + g], src_i)
            pltpu.sync_copy(e4_hbm.at[(NW + wid) * NG + g], dst_i)
            gather(0, rows0, sem0)

            def pair(t, _):
                j0 = t * 2
                gather(j0 + 1, rows1, sem1)
                pltpu.make_async_copy(table_hbm.at[src_i.at[j0]],
                                      rows0, sem0).wait()
                pltpu.sync_copy(rows0, acc.at[dst_i.at[j0]], add=True)

                @pl.when(t < NBUF // 2 - 1)
                def _():
                    gather(j0 + 2, rows0, sem0)
                pltpu.make_async_copy(table_hbm.at[src_i.at[j0 + 1]],
                                      rows1, sem1).wait()
                pltpu.sync_copy(rows1, acc.at[dst_i.at[j0 + 1]], add=True)
                return 0
            lax.fori_loop(0, NBUF // 2, pair, 0)
            return 0
        lax.fori_loop(0, NG, group, 0)

        plsc.subcore_barrier()

        pltpu.sync_copy(acc.at[pl.ds(base, SZ)], s_out.at[c, pl.ds(base, SZ)])

    return pl.kernel(body, out_type=outs[0], mesh=mesh,
                     scratch_types=scratch)


SCAN = 2000


def _make_hist_kernel():
    out = jax.ShapeDtypeStruct((NC * NP * D,), jnp.float32)
    scratch = [
        pltpu.VMEM((SCAN,), jnp.int32),
        pltpu.VMEM((NP,), jnp.float32),
        pltpu.VMEM((16 * SZ,), jnp.float32),
        pltpu.VMEM((16 * D,), jnp.float32),
        pltpu.VMEM_SHARED((NS * NP,), jnp.float32),
    ]
    mesh = plsc.VectorSubcoreMesh(core_axis_name="c", subcore_axis_name="s",
                                  num_cores=NC, num_subcores=NS)

    def body(dst_hbm, cnt_out, dbuf, hist, cbuf, obuf, stage):
        c = lax.axis_index("c")
        s = lax.axis_index("s")
        wid = s * NC + c
        ones_v = jnp.ones((16,), jnp.float32)

        def zh(i, _):
            hist[pl.ds(i * 16, 16)] = jnp.zeros((16,), jnp.float32)
            return 0
        lax.fori_loop(0, NP // 16, zh, 0)

        def hgroup(g, _):
            pltpu.sync_copy(dst_hbm.at[pl.ds(wid * EP + g * SCAN, SCAN)],
                            dbuf)

            def hstep(j, _):
                v = dbuf[pl.ds(j * 16, 16)]
                plsc.addupdate_scatter(hist, [v], ones_v)
                return 0
            lax.fori_loop(0, SCAN // 16, hstep, 0)
            return 0
        lax.fori_loop(0, EP // SCAN, hgroup, 0)

        pltpu.sync_copy(hist, stage.at[pl.ds(s * NP, NP)])
        plsc.subcore_barrier()

        lo = s * SZ
        for r in range(NS):
            pltpu.sync_copy(stage.at[pl.ds(r * NP + lo, SZ)],
                            cbuf.at[pl.ds(r * SZ, SZ)])

        def csum(m, _):
            acc16 = cbuf[pl.ds(m * 16, 16)]
            for r in range(1, NS):
                acc16 += cbuf[pl.ds(r * SZ + m * 16, 16)]
            hist[pl.ds(m * 16, 16)] = acc16
            return 0
        lax.fori_loop(0, SZ // 16, csum, 0)

        obase = c * NP * D + lo * D

        def ochunk(t, _):
            for rr in range(16):
                row = plsc.load_gather(
                    hist, [jnp.full((16,), t * 16 + rr, jnp.int32)])
                for q in range(D // 16):
                    obuf[pl.ds(rr * D + q * 16, 16)] = row
            pltpu.sync_copy(obuf, cnt_out.at[pl.ds(obase + t * 16 * D,
                                                   16 * D)])
            return 0
        lax.fori_loop(0, SZ // 16, ochunk, 0)

    return pl.kernel(
        body, out_type=out, mesh=mesh, scratch_types=scratch,
        compiler_params=pltpu.CompilerParams(needs_layout_passes=False))


@functools.lru_cache(maxsize=None)
def _seg_kernel():
    return _make_seg_kernel()


@functools.lru_cache(maxsize=None)
def _hist_kernel():
    return _make_hist_kernel()


def _lin_body(x_ref, w_ref, b_ref, o_ref):
    o_ref[...] = (lax.dot_general(x_ref[...], w_ref[...],
                                  (((1,), (1,)), ((), ())),
                                  preferred_element_type=jnp.float32)
                  + b_ref[...])


_lin_tc = pl.pallas_call(
    _lin_body,
    grid=(GRID,),
    in_specs=[
        pl.BlockSpec((BN, D), lambda i: (i, 0)),
        pl.BlockSpec((D, D), lambda i: (0, 0)),
        pl.BlockSpec((1, D), lambda i: (0, 0)),
    ],
    out_specs=pl.BlockSpec((BN, D), lambda i: (i, 0)),
    out_shape=jax.ShapeDtypeStruct((N, D), jnp.float32),
)


def _sage_body(s_ref, c_ref, y_ref, wl_ref, o_ref):
    ssum = s_ref[0] + s_ref[1]
    agg = ssum * (1.0 / jnp.maximum(c_ref[0] + c_ref[1], 1.0))
    h = (lax.dot_general(agg, wl_ref[...], (((1,), (1,)), ((), ())),
                         preferred_element_type=jnp.float32)
         + y_ref[...])
    o_ref[...] = jnp.maximum(h, 0.0)


_sage_tc = pl.pallas_call(
    _sage_body,
    grid=(GRID,),
    in_specs=[
        pl.BlockSpec((NC, BN, D), lambda i: (0, i, 0)),
        pl.BlockSpec((NC, BN, D), lambda i: (0, i, 0)),
        pl.BlockSpec((BN, D), lambda i: (i, 0)),
        pl.BlockSpec((D, D), lambda i: (0, 0)),
    ],
    out_specs=pl.BlockSpec((BN, D), lambda i: (i, 0)),
    out_shape=jax.ShapeDtypeStruct((N, D), jnp.float32),
)


def _head_body(s_ref, c_ref, y_ref, b_ref, wl_ref,
               wf1_ref, bf1_ref, wf2_ref, bf2_ref, o_ref,
               pooled_s, gcnt_s):
    i = pl.program_id(0)

    @pl.when(i == 0)
    def _():
        pooled_s[...] = jnp.zeros_like(pooled_s)
        gcnt_s[...] = jnp.zeros_like(gcnt_s)

    ssum = s_ref[0] + s_ref[1]
    agg = ssum * (1.0 / jnp.maximum(c_ref[0] + c_ref[1], 1.0))
    h2 = (lax.dot_general(agg, wl_ref[...], (((1,), (1,)), ((), ())),
                          preferred_element_type=jnp.float32)
          + y_ref[...])
    h2 = jnp.maximum(h2, 0.0)

    b = b_ref[0, 0]
    gids = lax.broadcasted_iota(jnp.int32, (BN, G), 1)
    onehot = (b[:, None] == gids).astype(jnp.float32)
    pooled_s[...] += lax.dot_general(onehot, h2, (((0,), (0,)), ((), ())),
                                     preferred_element_type=jnp.float32)
    gcnt_s[...] += lax.dot_general(onehot, jnp.ones((BN, D), jnp.float32),
                                   (((0,), (0,)), ((), ())),
                                   preferred_element_type=jnp.float32)

    @pl.when(i == GRID - 1)
    def _():
        pooled = pooled_s[...] * (1.0 / jnp.maximum(gcnt_s[...], 1.0))
        t = jnp.maximum(
            lax.dot_general(pooled, wf1_ref[...], (((1,), (1,)), ((), ())),
                            preferred_element_type=jnp.float32)
            + bf1_ref[...], 0.0)
        o_ref[...] = (lax.dot_general(wf2_ref[...], t, (((1,), (1,)), ((), ())),
                                      preferred_element_type=jnp.float32)
                      + bf2_ref[0, 0])


_head_tc = pl.pallas_call(
    _head_body,
    grid=(GRID,),
    in_specs=[
        pl.BlockSpec((NC, BN, D), lambda i: (0, i, 0)),
        pl.BlockSpec((NC, BN, D), lambda i: (0, i, 0)),
        pl.BlockSpec((BN, D), lambda i: (i, 0)),
        pl.BlockSpec((1, 1, BN), lambda i: (i, 0, 0)),
        pl.BlockSpec((D, D), lambda i: (0, 0)),
        pl.BlockSpec((H_FC, D), lambda i: (0, 0)),
        pl.BlockSpec((1, H_FC), lambda i: (0, 0)),
        pl.BlockSpec((1, H_FC), lambda i: (0, 0)),
        pl.BlockSpec((1, 1), lambda i: (0, 0)),
    ],
    out_specs=pl.BlockSpec((1, G), lambda i: (0, 0)),
    out_shape=jax.ShapeDtypeStruct((1, G), jnp.float32),
    scratch_shapes=[
        pltpu.VMEM((G, D), jnp.float32),
        pltpu.VMEM((G, D), jnp.float32),
    ],
)


def kernel(x, edge_index, batch, W1l, b1l, W1r, W2l, b2l, W2r,
           Wf1, bf1, Wf2, bf2):
    e4 = edge_index.reshape(2 * NW * NG, NBUF, CH)
    cnt = _hist_kernel()(edge_index[1]).reshape(NC, NP, D)
    s1 = _seg_kernel()(x, e4)
    y1 = _lin_tc(x, W1r, b1l.reshape(1, D))
    h1 = _sage_tc(s1, cnt, y1, W1l)
    s2 = _seg_kernel()(h1, e4)
    y2 = _lin_tc(h1, W2r, b2l.reshape(1, D))
    out = _head_tc(s2, cnt, y2, batch.reshape(GRID, 1, BN), W2l,
                   Wf1, bf1.reshape(1, H_FC), Wf2, bf2.reshape(1, 1))
    return out.reshape(-1)


# speedup vs baseline: 11.7820x; 1.0413x over previous
import functools

import jax
import jax.numpy as jnp
from jax import lax
from jax.experimental import pallas as pl
from jax.experimental.pallas import tpu as pltpu
from jax.experimental.pallas import tpu_sc as plsc

N, E, D, G, H_FC = 10000, 320000, 128, 64, 64
NC, NS = 2, 16
NW = NC * NS
EP = E // NW
CH = 125
NCH = EP // CH
NBUF = 16
NG = NCH // NBUF
NP = 10240
SZ = NP // NS
RP = NP // NW
BN = 1000
GRID = N // BN


def _make_seg_kernel():
    outs = [jax.ShapeDtypeStruct((NC, NP, D), jnp.float32)]
    scratch = [
        pltpu.VMEM((2, NBUF, CH), jnp.int32),
        pltpu.VMEM((2, NBUF, CH), jnp.int32),
        pltpu.VMEM((CH, D), jnp.float32),
        pltpu.VMEM((CH, D), jnp.float32),
        pltpu.VMEM_SHARED((NP, D), jnp.float32),
        pltpu.SemaphoreType.DMA,
        pltpu.SemaphoreType.DMA,
        pltpu.SemaphoreType.DMA,
    ]
    mesh = plsc.VectorSubcoreMesh(core_axis_name="c", subcore_axis_name="s",
                                  num_cores=NC, num_subcores=NS)

    def body(table_hbm, e4_hbm, *rest):
        (s_out, src_i, dst_i, rows0, rows1, acc, sem0, sem1, semi) = rest
        c = lax.axis_index("c")
        s = lax.axis_index("s")
        wid = s * NC + c
        base = s * SZ

        def prefetch(g, p):
            pltpu.async_copy(e4_hbm.at[wid * NG + g], src_i.at[p], semi)
            pltpu.async_copy(e4_hbm.at[(NW + wid) * NG + g], dst_i.at[p],
                             semi)

        def prefetch_wait(g, p):
            pltpu.make_async_copy(e4_hbm.at[wid * NG + g], src_i.at[p],
                                  semi).wait()
            pltpu.make_async_copy(e4_hbm.at[(NW + wid) * NG + g],
                                  dst_i.at[p], semi).wait()

        prefetch(0, 0)

        def zr(i, _):
            rows0[i // 8, pl.ds((i % 8) * 16, 16)] = jnp.zeros((16,),
                                                               jnp.float32)
            return 0
        lax.fori_loop(0, CH * (D // 16), zr, 0)
        for j in range(SZ // 80):
            pltpu.sync_copy(rows0.at[pl.ds(0, 80)],
                            acc.at[pl.ds(base + j * 80, 80)])
        plsc.subcore_barrier()

        prefetch_wait(0, 0)

        def gather(p, j, rbuf, sem):
            return pltpu.async_copy(table_hbm.at[src_i.at[p, j]], rbuf, sem)

        def group(g, _):
            p = g % 2
            gather(p, 0, rows0, sem0)

            @pl.when(g < NG - 1)
            def _():
                prefetch(g + 1, 1 - p)

            def pair(t, _):
                j0 = t * 2
                gather(p, j0 + 1, rows1, sem1)
                pltpu.make_async_copy(table_hbm.at[src_i.at[p, j0]],
                                      rows0, sem0).wait()
                pltpu.sync_copy(rows0, acc.at[dst_i.at[p, j0]], add=True)

                @pl.when(t < NBUF // 2 - 1)
                def _():
                    gather(p, j0 + 2, rows0, sem0)
                pltpu.make_async_copy(table_hbm.at[src_i.at[p, j0 + 1]],
                                      rows1, sem1).wait()
                pltpu.sync_copy(rows1, acc.at[dst_i.at[p, j0 + 1]], add=True)
                return 0
            lax.fori_loop(0, NBUF // 2, pair, 0)

            @pl.when(g < NG - 1)
            def _():
                prefetch_wait(g + 1, 1 - p)
            return 0
        lax.fori_loop(0, NG, group, 0)

        plsc.subcore_barrier()

        pltpu.sync_copy(acc.at[pl.ds(base, SZ)], s_out.at[c, pl.ds(base, SZ)])

    return pl.kernel(body, out_type=outs[0], mesh=mesh,
                     scratch_types=scratch)


SCAN = 2000


def _make_hist_kernel():
    out = jax.ShapeDtypeStruct((NC * NP * D,), jnp.float32)
    scratch = [
        pltpu.VMEM((SCAN,), jnp.int32),
        pltpu.VMEM((NP,), jnp.float32),
        pltpu.VMEM((16 * SZ,), jnp.float32),
        pltpu.VMEM((16 * D,), jnp.float32),
        pltpu.VMEM_SHARED((NS * NP,), jnp.float32),
    ]
    mesh = plsc.VectorSubcoreMesh(core_axis_name="c", subcore_axis_name="s",
                                  num_cores=NC, num_subcores=NS)

    def body(dst_hbm, cnt_out, dbuf, hist, cbuf, obuf, stage):
        c = lax.axis_index("c")
        s = lax.axis_index("s")
        wid = s * NC + c
        ones_v = jnp.ones((16,), jnp.float32)

        def zh(i, _):
            hist[pl.ds(i * 16, 16)] = jnp.zeros((16,), jnp.float32)
            return 0
        lax.fori_loop(0, NP // 16, zh, 0)

        def hgroup(g, _):
            pltpu.sync_copy(dst_hbm.at[pl.ds(wid * EP + g * SCAN, SCAN)],
                            dbuf)

            def hstep(j, _):
                v = dbuf[pl.ds(j * 16, 16)]
                plsc.addupdate_scatter(hist, [v], ones_v)
                return 0
            lax.fori_loop(0, SCAN // 16, hstep, 0)
            return 0
        lax.fori_loop(0, EP // SCAN, hgroup, 0)

        pltpu.sync_copy(hist, stage.at[pl.ds(s * NP, NP)])
        plsc.subcore_barrier()

        lo = s * SZ
        for r in range(NS):
            pltpu.sync_copy(stage.at[pl.ds(r * NP + lo, SZ)],
                            cbuf.at[pl.ds(r * SZ, SZ)])

        def csum(m, _):
            acc16 = cbuf[pl.ds(m * 16, 16)]
            for r in range(1, NS):
                acc16 += cbuf[pl.ds(r * SZ + m * 16, 16)]
            hist[pl.ds(m * 16, 16)] = acc16
            return 0
        lax.fori_loop(0, SZ // 16, csum, 0)

        obase = c * NP * D + lo * D

        def ochunk(t, _):
            for rr in range(16):
                row = plsc.load_gather(
                    hist, [jnp.full((16,), t * 16 + rr, jnp.int32)])
                for q in range(D // 16):
                    obuf[pl.ds(rr * D + q * 16, 16)] = row
            pltpu.sync_copy(obuf, cnt_out.at[pl.ds(obase + t * 16 * D,
                                                   16 * D)])
            return 0
        lax.fori_loop(0, SZ // 16, ochunk, 0)

    return pl.kernel(
        body, out_type=out, mesh=mesh, scratch_types=scratch,
        compiler_params=pltpu.CompilerParams(needs_layout_passes=False))


@functools.lru_cache(maxsize=None)
def _seg_kernel():
    return _make_seg_kernel()


@functools.lru_cache(maxsize=None)
def _hist_kernel():
    return _make_hist_kernel()


def _lin_body(x_ref, w_ref, b_ref, o_ref):
    o_ref[...] = (lax.dot_general(x_ref[...], w_ref[...],
                                  (((1,), (1,)), ((), ())),
                                  preferred_element_type=jnp.float32)
                  + b_ref[...])


_lin_tc = pl.pallas_call(
    _lin_body,
    grid=(GRID,),
    in_specs=[
        pl.BlockSpec((BN, D), lambda i: (i, 0)),
        pl.BlockSpec((D, D), lambda i: (0, 0)),
        pl.BlockSpec((1, D), lambda i: (0, 0)),
    ],
    out_specs=pl.BlockSpec((BN, D), lambda i: (i, 0)),
    out_shape=jax.ShapeDtypeStruct((N, D), jnp.float32),
)


def _sage_body(s_ref, c_ref, y_ref, wl_ref, o_ref):
    ssum = s_ref[0] + s_ref[1]
    agg = ssum * (1.0 / jnp.maximum(c_ref[0] + c_ref[1], 1.0))
    h = (lax.dot_general(agg, wl_ref[...], (((1,), (1,)), ((), ())),
                         preferred_element_type=jnp.float32)
         + y_ref[...])
    o_ref[...] = jnp.maximum(h, 0.0)


_sage_tc = pl.pallas_call(
    _sage_body,
    grid=(GRID,),
    in_specs=[
        pl.BlockSpec((NC, BN, D), lambda i: (0, i, 0)),
        pl.BlockSpec((NC, BN, D), lambda i: (0, i, 0)),
        pl.BlockSpec((BN, D), lambda i: (i, 0)),
        pl.BlockSpec((D, D), lambda i: (0, 0)),
    ],
    out_specs=pl.BlockSpec((BN, D), lambda i: (i, 0)),
    out_shape=jax.ShapeDtypeStruct((N, D), jnp.float32),
)


def _head_body(s_ref, c_ref, y_ref, b_ref, wl_ref,
               wf1_ref, bf1_ref, wf2_ref, bf2_ref, o_ref,
               pooled_s, gcnt_s):
    i = pl.program_id(0)

    @pl.when(i == 0)
    def _():
        pooled_s[...] = jnp.zeros_like(pooled_s)
        gcnt_s[...] = jnp.zeros_like(gcnt_s)

    ssum = s_ref[0] + s_ref[1]
    agg = ssum * (1.0 / jnp.maximum(c_ref[0] + c_ref[1], 1.0))
    h2 = (lax.dot_general(agg, wl_ref[...], (((1,), (1,)), ((), ())),
                          preferred_element_type=jnp.float32)
          + y_ref[...])
    h2 = jnp.maximum(h2, 0.0)

    b = b_ref[0, 0]
    gids = lax.broadcasted_iota(jnp.int32, (BN, G), 1)
    onehot = (b[:, None] == gids).astype(jnp.float32)
    pooled_s[...] += lax.dot_general(onehot, h2, (((0,), (0,)), ((), ())),
                                     preferred_element_type=jnp.float32)
    gcnt_s[...] += lax.dot_general(onehot, jnp.ones((BN, D), jnp.float32),
                                   (((0,), (0,)), ((), ())),
                                   preferred_element_type=jnp.float32)

    @pl.when(i == GRID - 1)
    def _():
        pooled = pooled_s[...] * (1.0 / jnp.maximum(gcnt_s[...], 1.0))
        t = jnp.maximum(
            lax.dot_general(pooled, wf1_ref[...], (((1,), (1,)), ((), ())),
                            preferred_element_type=jnp.float32)
            + bf1_ref[...], 0.0)
        o_ref[...] = (lax.dot_general(wf2_ref[...], t, (((1,), (1,)), ((), ())),
                                      preferred_element_type=jnp.float32)
                      + bf2_ref[0, 0])


_head_tc = pl.pallas_call(
    _head_body,
    grid=(GRID,),
    in_specs=[
        pl.BlockSpec((NC, BN, D), lambda i: (0, i, 0)),
        pl.BlockSpec((NC, BN, D), lambda i: (0, i, 0)),
        pl.BlockSpec((BN, D), lambda i: (i, 0)),
        pl.BlockSpec((1, 1, BN), lambda i: (i, 0, 0)),
        pl.BlockSpec((D, D), lambda i: (0, 0)),
        pl.BlockSpec((H_FC, D), lambda i: (0, 0)),
        pl.BlockSpec((1, H_FC), lambda i: (0, 0)),
        pl.BlockSpec((1, H_FC), lambda i: (0, 0)),
        pl.BlockSpec((1, 1), lambda i: (0, 0)),
    ],
    out_specs=pl.BlockSpec((1, G), lambda i: (0, 0)),
    out_shape=jax.ShapeDtypeStruct((1, G), jnp.float32),
    scratch_shapes=[
        pltpu.VMEM((G, D), jnp.float32),
        pltpu.VMEM((G, D), jnp.float32),
    ],
)


def kernel(x, edge_index, batch, W1l, b1l, W1r, W2l, b2l, W2r,
           Wf1, bf1, Wf2, bf2):
    e4 = edge_index.reshape(2 * NW * NG, NBUF, CH)
    cnt = _hist_kernel()(edge_index[1]).reshape(NC, NP, D)
    s1 = _seg_kernel()(x, e4)
    y1 = _lin_tc(x, W1r, b1l.reshape(1, D))
    h1 = _sage_tc(s1, cnt, y1, W1l)
    s2 = _seg_kernel()(h1, e4)
    y2 = _lin_tc(h1, W2r, b2l.reshape(1, D))
    out = _head_tc(s2, cnt, y2, batch.reshape(GRID, 1, BN), W2l,
                   Wf1, bf1.reshape(1, H_FC), Wf2, bf2.reshape(1, 1))
    return out.reshape(-1)


# speedup vs baseline: 12.0411x; 1.0220x over previous
import functools

import jax
import jax.numpy as jnp
from jax import lax
from jax.experimental import pallas as pl
from jax.experimental.pallas import tpu as pltpu
from jax.experimental.pallas import tpu_sc as plsc

N, E, D, G, H_FC = 10000, 320000, 128, 64, 64
NC, NS = 2, 16
NW = NC * NS
EP = E // NW
CH = 125
NCH = EP // CH
NBUF = 16
NG = NCH // NBUF
NP = 10240
SZ = NP // NS
RP = NP // NW
BN = 2000
GRID = N // BN


def _make_seg_kernel():
    outs = [jax.ShapeDtypeStruct((NC, NP, D), jnp.float32)]
    scratch = [
        pltpu.VMEM((2, NBUF, CH), jnp.int32),
        pltpu.VMEM((2, NBUF, CH), jnp.int32),
        pltpu.VMEM((CH, D), jnp.float32),
        pltpu.VMEM((CH, D), jnp.float32),
        pltpu.VMEM_SHARED((NP, D), jnp.float32),
        pltpu.SemaphoreType.DMA,
        pltpu.SemaphoreType.DMA,
        pltpu.SemaphoreType.DMA,
    ]
    mesh = plsc.VectorSubcoreMesh(core_axis_name="c", subcore_axis_name="s",
                                  num_cores=NC, num_subcores=NS)

    def body(table_hbm, e4_hbm, *rest):
        (s_out, src_i, dst_i, rows0, rows1, acc, sem0, sem1, semi) = rest
        c = lax.axis_index("c")
        s = lax.axis_index("s")
        wid = s * NC + c
        base = s * SZ

        def prefetch(g, p):
            pltpu.async_copy(e4_hbm.at[wid * NG + g], src_i.at[p], semi)
            pltpu.async_copy(e4_hbm.at[(NW + wid) * NG + g], dst_i.at[p],
                             semi)

        def prefetch_wait(g, p):
            pltpu.make_async_copy(e4_hbm.at[wid * NG + g], src_i.at[p],
                                  semi).wait()
            pltpu.make_async_copy(e4_hbm.at[(NW + wid) * NG + g],
                                  dst_i.at[p], semi).wait()

        prefetch(0, 0)

        def zr(i, _):
            rows0[i // 8, pl.ds((i % 8) * 16, 16)] = jnp.zeros((16,),
                                                               jnp.float32)
            return 0
        lax.fori_loop(0, CH * (D // 16), zr, 0)
        for j in range(SZ // 80):
            pltpu.sync_copy(rows0.at[pl.ds(0, 80)],
                            acc.at[pl.ds(base + j * 80, 80)])
        plsc.subcore_barrier()

        prefetch_wait(0, 0)

        def gather(p, j, rbuf, sem):
            return pltpu.async_copy(table_hbm.at[src_i.at[p, j]], rbuf, sem)

        def group(g, _):
            p = g % 2
            gather(p, 0, rows0, sem0)

            @pl.when(g < NG - 1)
            def _():
                prefetch(g + 1, 1 - p)

            def pair(t, _):
                j0 = t * 2
                gather(p, j0 + 1, rows1, sem1)
                pltpu.make_async_copy(table_hbm.at[src_i.at[p, j0]],
                                      rows0, sem0).wait()
                pltpu.sync_copy(rows0, acc.at[dst_i.at[p, j0]], add=True)

                @pl.when(t < NBUF // 2 - 1)
                def _():
                    gather(p, j0 + 2, rows0, sem0)
                pltpu.make_async_copy(table_hbm.at[src_i.at[p, j0 + 1]],
                                      rows1, sem1).wait()
                pltpu.sync_copy(rows1, acc.at[dst_i.at[p, j0 + 1]], add=True)
                return 0
            lax.fori_loop(0, NBUF // 2, pair, 0)

            @pl.when(g < NG - 1)
            def _():
                prefetch_wait(g + 1, 1 - p)
            return 0
        lax.fori_loop(0, NG, group, 0)

        plsc.subcore_barrier()

        pltpu.sync_copy(acc.at[pl.ds(base, SZ)], s_out.at[c, pl.ds(base, SZ)])

    return pl.kernel(body, out_type=outs[0], mesh=mesh,
                     scratch_types=scratch)


SCAN = 2000


def _make_hist_kernel():
    out = jax.ShapeDtypeStruct((NC * NP * D,), jnp.float32)
    scratch = [
        pltpu.VMEM((SCAN,), jnp.int32),
        pltpu.VMEM((NP,), jnp.float32),
        pltpu.VMEM((16 * SZ,), jnp.float32),
        pltpu.VMEM((16 * D,), jnp.float32),
        pltpu.VMEM_SHARED((NS * NP,), jnp.float32),
    ]
    mesh = plsc.VectorSubcoreMesh(core_axis_name="c", subcore_axis_name="s",
                                  num_cores=NC, num_subcores=NS)

    def body(dst_hbm, cnt_out, dbuf, hist, cbuf, obuf, stage):
        c = lax.axis_index("c")
        s = lax.axis_index("s")
        wid = s * NC + c
        ones_v = jnp.ones((16,), jnp.float32)

        def zh(i, _):
            hist[pl.ds(i * 16, 16)] = jnp.zeros((16,), jnp.float32)
            return 0
        lax.fori_loop(0, NP // 16, zh, 0)

        def hgroup(g, _):
            pltpu.sync_copy(dst_hbm.at[pl.ds(wid * EP + g * SCAN, SCAN)],
                            dbuf)

            def hstep(j, _):
                v = dbuf[pl.ds(j * 16, 16)]
                plsc.addupdate_scatter(hist, [v], ones_v)
                return 0
            lax.fori_loop(0, SCAN // 16, hstep, 0)
            return 0
        lax.fori_loop(0, EP // SCAN, hgroup, 0)

        pltpu.sync_copy(hist, stage.at[pl.ds(s * NP, NP)])
        plsc.subcore_barrier()

        lo = s * SZ
        for r in range(NS):
            pltpu.sync_copy(stage.at[pl.ds(r * NP + lo, SZ)],
                            cbuf.at[pl.ds(r * SZ, SZ)])

        def csum(m, _):
            acc16 = cbuf[pl.ds(m * 16, 16)]
            for r in range(1, NS):
                acc16 += cbuf[pl.ds(r * SZ + m * 16, 16)]
            hist[pl.ds(m * 16, 16)] = acc16
            return 0
        lax.fori_loop(0, SZ // 16, csum, 0)

        obase = c * NP * D + lo * D

        def ochunk(t, _):
            for rr in range(16):
                row = plsc.load_gather(
                    hist, [jnp.full((16,), t * 16 + rr, jnp.int32)])
                for q in range(D // 16):
                    obuf[pl.ds(rr * D + q * 16, 16)] = row
            pltpu.sync_copy(obuf, cnt_out.at[pl.ds(obase + t * 16 * D,
                                                   16 * D)])
            return 0
        lax.fori_loop(0, SZ // 16, ochunk, 0)

    return pl.kernel(
        body, out_type=out, mesh=mesh, scratch_types=scratch,
        compiler_params=pltpu.CompilerParams(needs_layout_passes=False))


@functools.lru_cache(maxsize=None)
def _seg_kernel():
    return _make_seg_kernel()


@functools.lru_cache(maxsize=None)
def _hist_kernel():
    return _make_hist_kernel()


def _sage_body(s_ref, c_ref, x_ref, wl_ref, bl_ref, wr_ref, o_ref):
    ssum = s_ref[0] + s_ref[1]
    agg = ssum * (1.0 / jnp.maximum(c_ref[0] + c_ref[1], 1.0))
    h = (lax.dot_general(agg, wl_ref[...], (((1,), (1,)), ((), ())),
                         preferred_element_type=jnp.float32)
         + bl_ref[...]
         + lax.dot_general(x_ref[...], wr_ref[...], (((1,), (1,)), ((), ())),
                           preferred_element_type=jnp.float32))
    o_ref[...] = jnp.maximum(h, 0.0)


_sage_tc = pl.pallas_call(
    _sage_body,
    grid=(GRID,),
    in_specs=[
        pl.BlockSpec((NC, BN, D), lambda i: (0, i, 0)),
        pl.BlockSpec((NC, BN, D), lambda i: (0, i, 0)),
        pl.BlockSpec((BN, D), lambda i: (i, 0)),
        pl.BlockSpec((D, D), lambda i: (0, 0)),
        pl.BlockSpec((1, D), lambda i: (0, 0)),
        pl.BlockSpec((D, D), lambda i: (0, 0)),
    ],
    out_specs=pl.BlockSpec((BN, D), lambda i: (i, 0)),
    out_shape=jax.ShapeDtypeStruct((N, D), jnp.float32),
)


def _head_body(s_ref, c_ref, x_ref, b_ref, wl_ref, bl_ref, wr_ref,
               wf1_ref, bf1_ref, wf2_ref, bf2_ref, o_ref,
               pooled_s, gcnt_s):
    i = pl.program_id(0)

    @pl.when(i == 0)
    def _():
        pooled_s[...] = jnp.zeros_like(pooled_s)
        gcnt_s[...] = jnp.zeros_like(gcnt_s)

    ssum = s_ref[0] + s_ref[1]
    agg = ssum * (1.0 / jnp.maximum(c_ref[0] + c_ref[1], 1.0))
    h2 = (lax.dot_general(agg, wl_ref[...], (((1,), (1,)), ((), ())),
                          preferred_element_type=jnp.float32)
          + bl_ref[...]
          + lax.dot_general(x_ref[...], wr_ref[...], (((1,), (1,)), ((), ())),
                            preferred_element_type=jnp.float32))
    h2 = jnp.maximum(h2, 0.0)

    b = b_ref[0, 0]
    gids = lax.broadcasted_iota(jnp.int32, (BN, G), 1)
    onehot = (b[:, None] == gids).astype(jnp.float32)
    pooled_s[...] += lax.dot_general(onehot, h2, (((0,), (0,)), ((), ())),
                                     preferred_element_type=jnp.float32)
    gcnt_s[...] += lax.dot_general(onehot, jnp.ones((BN, D), jnp.float32),
                                   (((0,), (0,)), ((), ())),
                                   preferred_element_type=jnp.float32)

    @pl.when(i == GRID - 1)
    def _():
        pooled = pooled_s[...] * (1.0 / jnp.maximum(gcnt_s[...], 1.0))
        t = jnp.maximum(
            lax.dot_general(pooled, wf1_ref[...], (((1,), (1,)), ((), ())),
                            preferred_element_type=jnp.float32)
            + bf1_ref[...], 0.0)
        o_ref[...] = (lax.dot_general(wf2_ref[...], t, (((1,), (1,)), ((), ())),
                                      preferred_element_type=jnp.float32)
                      + bf2_ref[0, 0])


_head_tc = pl.pallas_call(
    _head_body,
    grid=(GRID,),
    in_specs=[
        pl.BlockSpec((NC, BN, D), lambda i: (0, i, 0)),
        pl.BlockSpec((NC, BN, D), lambda i: (0, i, 0)),
        pl.BlockSpec((BN, D), lambda i: (i, 0)),
        pl.BlockSpec((1, 1, BN), lambda i: (i, 0, 0)),
        pl.BlockSpec((D, D), lambda i: (0, 0)),
        pl.BlockSpec((1, D), lambda i: (0, 0)),
        pl.BlockSpec((D, D), lambda i: (0, 0)),
        pl.BlockSpec((H_FC, D), lambda i: (0, 0)),
        pl.BlockSpec((1, H_FC), lambda i: (0, 0)),
        pl.BlockSpec((1, H_FC), lambda i: (0, 0)),
        pl.BlockSpec((1, 1), lambda i: (0, 0)),
    ],
    out_specs=pl.BlockSpec((1, G), lambda i: (0, 0)),
    out_shape=jax.ShapeDtypeStruct((1, G), jnp.float32),
    scratch_shapes=[
        pltpu.VMEM((G, D), jnp.float32),
        pltpu.VMEM((G, D), jnp.float32),
    ],
)


def kernel(x, edge_index, batch, W1l, b1l, W1r, W2l, b2l, W2r,
           Wf1, bf1, Wf2, bf2):
    e4 = edge_index.reshape(2 * NW * NG, NBUF, CH)
    cnt = _hist_kernel()(edge_index[1]).reshape(NC, NP, D)
    s1 = _seg_kernel()(x, e4)
    h1 = _sage_tc(s1, cnt, x, W1l, b1l.reshape(1, D), W1r)
    s2 = _seg_kernel()(h1, e4)
    out = _head_tc(s2, cnt, h1, batch.reshape(GRID, 1, BN),
                   W2l, b2l.reshape(1, D), W2r,
                   Wf1, bf1.reshape(1, H_FC), Wf2, bf2.reshape(1, 1))
    return out.reshape(-1)
